# initial kernel scaffold (unmeasured)
import jax
import jax.numpy as jnp
from jax import lax
from jax.experimental import pallas as pl
from jax.experimental.pallas import tpu as pltpu

N_DEV = 8


def kernel(x, w_mat):
    m_total, k_per = x.shape
    _, n = w_mat.shape
    m_per = m_total // N_DEV

    def body(x_ref, w_ref, out_ref, comm_ref, send_sems, recv_sems,
             amax_ref, amax_send_sems, amax_recv_sems):
        my = lax.axis_index("i")
        left = lax.rem(my + N_DEV - 1, N_DEV)
        right = lax.rem(my + 1, N_DEV)

        barrier_sem = pltpu.get_barrier_semaphore()
        for nbr in (left, right):
            pl.semaphore_signal(
                barrier_sem, inc=1,
                device_id=(nbr,), device_id_type=pl.DeviceIdType.MESH,
            )
        pl.semaphore_wait(barrier_sem, 2)

        prev_rdma = None
        y = None
        for s in range(N_DEV):
            d = lax.rem(my + 2 * N_DEV - 1 - s, N_DEV)
            slot = s % 2
            partial = jnp.dot(
                x_ref[pl.ds(d * m_per, m_per), :], w_ref[:, :],
                preferred_element_type=jnp.float32,
            )
            if s == 0:
                comm_ref[slot] = partial
            else:
                prev_rdma.wait()
                comm_ref[slot] = comm_ref[slot] + partial
            if s < N_DEV - 1:
                rdma = pltpu.make_async_remote_copy(
                    src_ref=comm_ref.at[slot],
                    dst_ref=comm_ref.at[(s + 1) % 2],
                    send_sem=send_sems.at[slot],
                    recv_sem=recv_sems.at[(s + 1) % 2],
                    device_id=(right,),
                    device_id_type=pl.DeviceIdType.MESH,
                )
                rdma.start()
                prev_rdma = rdma
            else:
                y = comm_ref[slot]

        local_amax = jnp.max(jnp.abs(y))
        amax_ref[pl.ds(my * 8, 8), :] = jnp.full((8, 128), local_amax,
                                                 dtype=jnp.float32)
        amax_rdmas = []
        for k in range(1, N_DEV):
            peer = lax.rem(my + k, N_DEV)
            rd = pltpu.make_async_remote_copy(
                src_ref=amax_ref.at[pl.ds(my * 8, 8), :],
                dst_ref=amax_ref.at[pl.ds(my * 8, 8), :],
                send_sem=amax_send_sems.at[k - 1],
                recv_sem=amax_recv_sems.at[k - 1],
                device_id=(peer,),
                device_id_type=pl.DeviceIdType.MESH,
            )
            rd.start()
            amax_rdmas.append(rd)
        for rd in amax_rdmas:
            rd.wait()
        global_amax = jnp.max(amax_ref[:, :])

        scale = global_amax / 127.0
        q = jnp.clip(jnp.round(y / scale), -127.0, 127.0)
        out_ref[:, :] = q * scale

    return pl.pallas_call(
        body,
        out_shape=jax.ShapeDtypeStruct((m_per, n), jnp.float32),
        in_specs=[
            pl.BlockSpec(memory_space=pltpu.VMEM),
            pl.BlockSpec(memory_space=pltpu.VMEM),
        ],
        out_specs=pl.BlockSpec(memory_space=pltpu.VMEM),
        scratch_shapes=[
            pltpu.VMEM((2, m_per, n), jnp.float32),
            pltpu.SemaphoreType.DMA((2,)),
            pltpu.SemaphoreType.DMA((2,)),
            pltpu.VMEM((8 * N_DEV, 128), jnp.float32),
            pltpu.SemaphoreType.DMA((N_DEV - 1,)),
            pltpu.SemaphoreType.DMA((N_DEV - 1,)),
        ],
        compiler_params=pltpu.CompilerParams(collective_id=0),
    )(x, w_mat)


# baseline (device time: 348002 ns/iter reference)
import jax
import jax.numpy as jnp
from jax import lax
from jax.experimental import pallas as pl
from jax.experimental.pallas import tpu as pltpu

N_DEV = 8


def kernel(x, w_mat):
    m_total, k_per = x.shape
    _, n = w_mat.shape
    m_per = m_total // N_DEV

    def body(x_ref, w_ref, out_ref, comm_ref, send_sems, recv_sems,
             amax_ref, amax_recv_ref, amax_send_sems, amax_recv_sems):
        my = lax.axis_index("i")
        left = lax.rem(my + N_DEV - 1, N_DEV)
        right = lax.rem(my + 1, N_DEV)

        barrier_sem = pltpu.get_barrier_semaphore()
        for nbr in (left, right):
            pl.semaphore_signal(
                barrier_sem, inc=1,
                device_id=(nbr,), device_id_type=pl.DeviceIdType.MESH,
            )
        pl.semaphore_wait(barrier_sem, 2)

        prev_rdma = None
        y = None
        for s in range(N_DEV):
            d = lax.rem(my + 2 * N_DEV - 1 - s, N_DEV)
            slot = s % 2
            partial = jnp.dot(
                x_ref[pl.ds(d * m_per, m_per), :], w_ref[:, :],
                preferred_element_type=jnp.float32,
            )
            if s == 0:
                comm_ref[slot] = partial
            else:
                prev_rdma.wait()
                comm_ref[slot] = comm_ref[slot] + partial
            if s < N_DEV - 1:
                rdma = pltpu.make_async_remote_copy(
                    src_ref=comm_ref.at[slot],
                    dst_ref=comm_ref.at[(s + 1) % 2],
                    send_sem=send_sems.at[slot],
                    recv_sem=recv_sems.at[(s + 1) % 2],
                    device_id=(right,),
                    device_id_type=pl.DeviceIdType.MESH,
                )
                rdma.start()
                prev_rdma = rdma
            else:
                y = comm_ref[slot]

        local_amax = jnp.max(jnp.abs(y))
        amax_ref[:, :] = jnp.full((8, 128), local_amax, dtype=jnp.float32)
        amax_rdmas = []
        for k in range(1, N_DEV):
            peer = lax.rem(my + k, N_DEV)
            rd = pltpu.make_async_remote_copy(
                src_ref=amax_ref.at[pl.ds(0, 8), :],
                dst_ref=amax_recv_ref.at[pl.ds((k - 1) * 8, 8), :],
                send_sem=amax_send_sems.at[k - 1],
                recv_sem=amax_recv_sems.at[k - 1],
                device_id=(peer,),
                device_id_type=pl.DeviceIdType.MESH,
            )
            rd.start()
            amax_rdmas.append(rd)
        for rd in amax_rdmas:
            rd.wait()
        global_amax = jnp.maximum(local_amax, jnp.max(amax_recv_ref[:, :]))

        import os
        if os.environ.get("DEBUG_AMAX") == "1":
            out_ref[:, :] = jnp.full((m_per, n), global_amax, jnp.float32)
        else:
            scale = global_amax / 127.0
            q = jnp.clip(jnp.round(y / scale), -127.0, 127.0)
            out_ref[:, :] = q * scale

    return pl.pallas_call(
        body,
        out_shape=jax.ShapeDtypeStruct((m_per, n), jnp.float32),
        in_specs=[
            pl.BlockSpec(memory_space=pltpu.VMEM),
            pl.BlockSpec(memory_space=pltpu.VMEM),
        ],
        out_specs=pl.BlockSpec(memory_space=pltpu.VMEM),
        scratch_shapes=[
            pltpu.VMEM((2, m_per, n), jnp.float32),
            pltpu.SemaphoreType.DMA((2,)),
            pltpu.SemaphoreType.DMA((2,)),
            pltpu.VMEM((8, 128), jnp.float32),
            pltpu.VMEM((8 * (N_DEV - 1), 128), jnp.float32),
            pltpu.SemaphoreType.DMA((N_DEV - 1,)),
            pltpu.SemaphoreType.DMA((N_DEV - 1,)),
        ],
        compiler_params=pltpu.CompilerParams(collective_id=0),
    )(x, w_mat)


# device time: 198142 ns/iter; 1.7563x vs baseline; 1.7563x over previous
import jax
import jax.numpy as jnp
from jax import lax
from jax.experimental import pallas as pl
from jax.experimental.pallas import tpu as pltpu

N_DEV = 8


def kernel(x, w_mat):
    m_total, k_per = x.shape
    _, n = w_mat.shape
    m_per = m_total // N_DEV
    nh = n // 2

    def body(x_ref, w_ref, out_ref, commR, commL,
             sendR, recvR, sendL, recvL,
             amax_ref, amax_recv_ref, amax_send_sems, amax_recv_sems):
        my = lax.axis_index("i")
        left = lax.rem(my + N_DEV - 1, N_DEV)
        right = lax.rem(my + 1, N_DEV)

        barrier_sem = pltpu.get_barrier_semaphore()
        for nbr in (left, right):
            pl.semaphore_signal(
                barrier_sem, inc=1,
                device_id=(nbr,), device_id_type=pl.DeviceIdType.MESH,
            )
        pl.semaphore_wait(barrier_sem, 2)

        prevR = prevL = None
        for s in range(N_DEV):
            slot = s % 2
            d_R = lax.rem(my + 2 * N_DEV - 1 - s, N_DEV)
            d_L = lax.rem(my + 1 + s, N_DEV)
            pR = jnp.dot(
                x_ref[pl.ds(d_R * m_per, m_per), :], w_ref[:, nh:],
                preferred_element_type=jnp.float32,
                precision=lax.Precision.HIGHEST,
            )
            pL = jnp.dot(
                x_ref[pl.ds(d_L * m_per, m_per), :], w_ref[:, :nh],
                preferred_element_type=jnp.float32,
                precision=lax.Precision.HIGHEST,
            )
            if s == 0:
                commR[slot] = pR
                commL[slot] = pL
            else:
                prevR.wait()
                commR[slot] = commR[slot] + pR
                prevL.wait()
                commL[slot] = commL[slot] + pL
            if s < N_DEV - 1:
                prevR = pltpu.make_async_remote_copy(
                    src_ref=commR.at[slot],
                    dst_ref=commR.at[(s + 1) % 2],
                    send_sem=sendR.at[slot],
                    recv_sem=recvR.at[(s + 1) % 2],
                    device_id=(right,),
                    device_id_type=pl.DeviceIdType.MESH,
                )
                prevL = pltpu.make_async_remote_copy(
                    src_ref=commL.at[slot],
                    dst_ref=commL.at[(s + 1) % 2],
                    send_sem=sendL.at[slot],
                    recv_sem=recvL.at[(s + 1) % 2],
                    device_id=(left,),
                    device_id_type=pl.DeviceIdType.MESH,
                )
                prevR.start()
                prevL.start()
            else:
                yL = commL[slot]
                yR = commR[slot]

        local_amax = jnp.maximum(jnp.max(jnp.abs(yL)), jnp.max(jnp.abs(yR)))
        amax_ref[:, :] = jnp.full((8, 128), local_amax, dtype=jnp.float32)
        amax_rdmas = []
        for k in range(1, N_DEV):
            peer = lax.rem(my + k, N_DEV)
            rd = pltpu.make_async_remote_copy(
                src_ref=amax_ref.at[pl.ds(0, 8), :],
                dst_ref=amax_recv_ref.at[pl.ds((k - 1) * 8, 8), :],
                send_sem=amax_send_sems.at[k - 1],
                recv_sem=amax_recv_sems.at[k - 1],
                device_id=(peer,),
                device_id_type=pl.DeviceIdType.MESH,
            )
            rd.start()
            amax_rdmas.append(rd)
        for rd in amax_rdmas:
            rd.wait()
        global_amax = jnp.maximum(local_amax, jnp.max(amax_recv_ref[:, :]))

        scale = global_amax / 127.0
        qL = jnp.clip(jnp.round(yL / scale), -127.0, 127.0)
        qR = jnp.clip(jnp.round(yR / scale), -127.0, 127.0)
        out_ref[:, :nh] = qL * scale
        out_ref[:, nh:] = qR * scale

    return pl.pallas_call(
        body,
        out_shape=jax.ShapeDtypeStruct((m_per, n), jnp.float32),
        in_specs=[
            pl.BlockSpec(memory_space=pltpu.VMEM),
            pl.BlockSpec(memory_space=pltpu.VMEM),
        ],
        out_specs=pl.BlockSpec(memory_space=pltpu.VMEM),
        scratch_shapes=[
            pltpu.VMEM((2, m_per, nh), jnp.float32),
            pltpu.VMEM((2, m_per, nh), jnp.float32),
            pltpu.SemaphoreType.DMA((2,)),
            pltpu.SemaphoreType.DMA((2,)),
            pltpu.SemaphoreType.DMA((2,)),
            pltpu.SemaphoreType.DMA((2,)),
            pltpu.VMEM((8, 128), jnp.float32),
            pltpu.VMEM((8 * (N_DEV - 1), 128), jnp.float32),
            pltpu.SemaphoreType.DMA((N_DEV - 1,)),
            pltpu.SemaphoreType.DMA((N_DEV - 1,)),
        ],
        compiler_params=pltpu.CompilerParams(collective_id=0),
    )(x, w_mat)


# device time: 114871 ns/iter; 3.0295x vs baseline; 1.7249x over previous
import jax
import jax.numpy as jnp
from jax import lax
from jax.experimental import pallas as pl
from jax.experimental.pallas import tpu as pltpu

N_DEV = 8


def kernel(x, w_mat):
    m_total, k_per = x.shape
    _, n = w_mat.shape
    m_per = m_total // N_DEV
    nh = n // 2

    def body(x_ref, w_ref, out_ref, commR, commL,
             sendR, recvR, sendL, recvL,
             amax_ref, amax_recv_ref, amax_send_sems, amax_recv_sems):
        my = lax.axis_index("i")
        left = lax.rem(my + N_DEV - 1, N_DEV)
        right = lax.rem(my + 1, N_DEV)

        barrier_sem = pltpu.get_barrier_semaphore()
        for nbr in (left, right):
            pl.semaphore_signal(
                barrier_sem, inc=1,
                device_id=(nbr,), device_id_type=pl.DeviceIdType.MESH,
            )
        pl.semaphore_wait(barrier_sem, 2)

        w_hi = w_ref[:, :].astype(jnp.bfloat16)
        w_lo = (w_ref[:, :] - w_hi.astype(jnp.float32)).astype(jnp.bfloat16)

        def dot3(x_blk, wh, wl):
            xh = x_blk.astype(jnp.bfloat16)
            xl = (x_blk - xh.astype(jnp.float32)).astype(jnp.bfloat16)
            acc = jnp.dot(xh, wh, preferred_element_type=jnp.float32)
            acc += jnp.dot(xh, wl, preferred_element_type=jnp.float32)
            acc += jnp.dot(xl, wh, preferred_element_type=jnp.float32)
            return acc

        prevR = prevL = None
        for s in range(N_DEV):
            slot = s % 2
            d_R = lax.rem(my + 2 * N_DEV - 1 - s, N_DEV)
            d_L = lax.rem(my + 1 + s, N_DEV)
            pR = dot3(x_ref[pl.ds(d_R * m_per, m_per), :], w_hi[:, nh:],
                      w_lo[:, nh:])
            pL = dot3(x_ref[pl.ds(d_L * m_per, m_per), :], w_hi[:, :nh],
                      w_lo[:, :nh])
            if s == 0:
                commR[slot] = pR.astype(jnp.bfloat16)
                commL[slot] = pL.astype(jnp.bfloat16)
            elif s < N_DEV - 1:
                prevR.wait()
                commR[slot] = (commR[slot].astype(jnp.float32)
                               + pR).astype(jnp.bfloat16)
                prevL.wait()
                commL[slot] = (commL[slot].astype(jnp.float32)
                               + pL).astype(jnp.bfloat16)
            else:
                prevR.wait()
                yR = commR[slot].astype(jnp.float32) + pR
                prevL.wait()
                yL = commL[slot].astype(jnp.float32) + pL
            if s < N_DEV - 1:
                prevR = pltpu.make_async_remote_copy(
                    src_ref=commR.at[slot],
                    dst_ref=commR.at[(s + 1) % 2],
                    send_sem=sendR.at[slot],
                    recv_sem=recvR.at[(s + 1) % 2],
                    device_id=(right,),
                    device_id_type=pl.DeviceIdType.MESH,
                )
                prevL = pltpu.make_async_remote_copy(
                    src_ref=commL.at[slot],
                    dst_ref=commL.at[(s + 1) % 2],
                    send_sem=sendL.at[slot],
                    recv_sem=recvL.at[(s + 1) % 2],
                    device_id=(left,),
                    device_id_type=pl.DeviceIdType.MESH,
                )
                prevR.start()
                prevL.start()

        local_amax = jnp.maximum(jnp.max(jnp.abs(yL)), jnp.max(jnp.abs(yR)))
        amax_ref[:, :] = jnp.full((8, 128), local_amax, dtype=jnp.float32)
        amax_rdmas = []
        for k in range(1, N_DEV):
            peer = lax.rem(my + k, N_DEV)
            rd = pltpu.make_async_remote_copy(
                src_ref=amax_ref.at[pl.ds(0, 8), :],
                dst_ref=amax_recv_ref.at[pl.ds((k - 1) * 8, 8), :],
                send_sem=amax_send_sems.at[k - 1],
                recv_sem=amax_recv_sems.at[k - 1],
                device_id=(peer,),
                device_id_type=pl.DeviceIdType.MESH,
            )
            rd.start()
            amax_rdmas.append(rd)
        for rd in amax_rdmas:
            rd.wait()
        global_amax = jnp.maximum(local_amax, jnp.max(amax_recv_ref[:, :]))

        scale = global_amax / 127.0
        qL = jnp.clip(jnp.round(yL / scale), -127.0, 127.0)
        qR = jnp.clip(jnp.round(yR / scale), -127.0, 127.0)
        out_ref[:, :nh] = qL * scale
        out_ref[:, nh:] = qR * scale

    return pl.pallas_call(
        body,
        out_shape=jax.ShapeDtypeStruct((m_per, n), jnp.float32),
        in_specs=[
            pl.BlockSpec(memory_space=pltpu.VMEM),
            pl.BlockSpec(memory_space=pltpu.VMEM),
        ],
        out_specs=pl.BlockSpec(memory_space=pltpu.VMEM),
        scratch_shapes=[
            pltpu.VMEM((2, m_per, nh), jnp.bfloat16),
            pltpu.VMEM((2, m_per, nh), jnp.bfloat16),
            pltpu.SemaphoreType.DMA((2,)),
            pltpu.SemaphoreType.DMA((2,)),
            pltpu.SemaphoreType.DMA((2,)),
            pltpu.SemaphoreType.DMA((2,)),
            pltpu.VMEM((8, 128), jnp.float32),
            pltpu.VMEM((8 * (N_DEV - 1), 128), jnp.float32),
            pltpu.SemaphoreType.DMA((N_DEV - 1,)),
            pltpu.SemaphoreType.DMA((N_DEV - 1,)),
        ],
        compiler_params=pltpu.CompilerParams(collective_id=0),
    )(x, w_mat)


# device time: 100586 ns/iter; 3.4597x vs baseline; 1.1420x over previous
import jax
import jax.numpy as jnp
from jax import lax
from jax.experimental import pallas as pl
from jax.experimental.pallas import tpu as pltpu

N_DEV = 8


def kernel(x, w_mat):
    m_total, k_per = x.shape
    _, n = w_mat.shape
    m_per = m_total // N_DEV
    nh = n // 2

    def body(x_ref, w_ref, out_ref, commR, commL,
             sendR, recvR, sendL, recvL,
             amax_ref, amax_recv_ref, amax_send_sems, amax_recv_sems):
        my = lax.axis_index("i")
        left = lax.rem(my + N_DEV - 1, N_DEV)
        right = lax.rem(my + 1, N_DEV)

        barrier_sem = pltpu.get_barrier_semaphore()
        for nbr in (left, right):
            pl.semaphore_signal(
                barrier_sem, inc=1,
                device_id=(nbr,), device_id_type=pl.DeviceIdType.MESH,
            )
        pl.semaphore_wait(barrier_sem, 2)

        w_hi = w_ref[:, :].astype(jnp.bfloat16)
        w_lo = (w_ref[:, :] - w_hi.astype(jnp.float32)).astype(jnp.bfloat16)

        def dot3(x_blk, wh, wl):
            xh = x_blk.astype(jnp.bfloat16)
            xl = (x_blk - xh.astype(jnp.float32)).astype(jnp.bfloat16)
            acc = jnp.dot(xh, wh, preferred_element_type=jnp.float32)
            acc += jnp.dot(xh, wl, preferred_element_type=jnp.float32)
            acc += jnp.dot(xl, wh, preferred_element_type=jnp.float32)
            return acc

        nq = nh // 2

        def ring_rdma(comm, sems_send, sems_recv, s, c, nbr):
            return pltpu.make_async_remote_copy(
                src_ref=comm.at[s % 2, :, pl.ds(c * nq, nq)],
                dst_ref=comm.at[(s + 1) % 2, :, pl.ds(c * nq, nq)],
                send_sem=sems_send.at[s % 2, c],
                recv_sem=sems_recv.at[(s + 1) % 2, c],
                device_id=(nbr,),
                device_id_type=pl.DeviceIdType.MESH,
            )

        prevR = prevL = None
        yR = yL = None
        for s in range(N_DEV):
            slot = s % 2
            d_R = lax.rem(my + 2 * N_DEV - 1 - s, N_DEV)
            d_L = lax.rem(my + 1 + s, N_DEV)
            pR = dot3(x_ref[pl.ds(d_R * m_per, m_per), :], w_hi[:, nh:],
                      w_lo[:, nh:])
            pL = dot3(x_ref[pl.ds(d_L * m_per, m_per), :], w_hi[:, :nh],
                      w_lo[:, :nh])
            if s == 0:
                commR[slot] = pR.astype(jnp.bfloat16)
                commL[slot] = pL.astype(jnp.bfloat16)
                prevR = [ring_rdma(commR, sendR, recvR, s, c, right)
                         for c in (0, 1)]
                prevL = [ring_rdma(commL, sendL, recvL, s, c, left)
                         for c in (0, 1)]
                for rd in (prevR[0], prevL[0], prevR[1], prevL[1]):
                    rd.start()
            elif s < N_DEV - 1:
                nextR = [ring_rdma(commR, sendR, recvR, s, c, right)
                         for c in (0, 1)]
                nextL = [ring_rdma(commL, sendL, recvL, s, c, left)
                         for c in (0, 1)]
                for c in (0, 1):
                    cols = pl.ds(c * nq, nq)
                    prevR[c].wait()
                    commR[slot, :, cols] = (
                        commR[slot, :, cols].astype(jnp.float32)
                        + pR[:, c * nq:(c + 1) * nq]).astype(jnp.bfloat16)
                    nextR[c].start()
                    prevL[c].wait()
                    commL[slot, :, cols] = (
                        commL[slot, :, cols].astype(jnp.float32)
                        + pL[:, c * nq:(c + 1) * nq]).astype(jnp.bfloat16)
                    nextL[c].start()
                prevR, prevL = nextR, nextL
            else:
                prevR[0].wait()
                prevR[1].wait()
                yR = commR[slot].astype(jnp.float32) + pR
                prevL[0].wait()
                prevL[1].wait()
                yL = commL[slot].astype(jnp.float32) + pL

        local_amax = jnp.maximum(jnp.max(jnp.abs(yL)), jnp.max(jnp.abs(yR)))
        amax_ref[:, :] = jnp.full((8, 128), local_amax, dtype=jnp.float32)
        amax_rdmas = []
        for k in range(1, N_DEV):
            peer = lax.rem(my + k, N_DEV)
            rd = pltpu.make_async_remote_copy(
                src_ref=amax_ref.at[pl.ds(0, 8), :],
                dst_ref=amax_recv_ref.at[pl.ds((k - 1) * 8, 8), :],
                send_sem=amax_send_sems.at[k - 1],
                recv_sem=amax_recv_sems.at[k - 1],
                device_id=(peer,),
                device_id_type=pl.DeviceIdType.MESH,
            )
            rd.start()
            amax_rdmas.append(rd)
        for rd in amax_rdmas:
            rd.wait()
        global_amax = jnp.maximum(local_amax, jnp.max(amax_recv_ref[:, :]))

        scale = global_amax / 127.0
        qL = jnp.clip(jnp.round(yL / scale), -127.0, 127.0)
        qR = jnp.clip(jnp.round(yR / scale), -127.0, 127.0)
        out_ref[:, :nh] = qL * scale
        out_ref[:, nh:] = qR * scale

    return pl.pallas_call(
        body,
        out_shape=jax.ShapeDtypeStruct((m_per, n), jnp.float32),
        in_specs=[
            pl.BlockSpec(memory_space=pltpu.VMEM),
            pl.BlockSpec(memory_space=pltpu.VMEM),
        ],
        out_specs=pl.BlockSpec(memory_space=pltpu.VMEM),
        scratch_shapes=[
            pltpu.VMEM((2, m_per, nh), jnp.bfloat16),
            pltpu.VMEM((2, m_per, nh), jnp.bfloat16),
            pltpu.SemaphoreType.DMA((2, 2)),
            pltpu.SemaphoreType.DMA((2, 2)),
            pltpu.SemaphoreType.DMA((2, 2)),
            pltpu.SemaphoreType.DMA((2, 2)),
            pltpu.VMEM((8, 128), jnp.float32),
            pltpu.VMEM((8 * (N_DEV - 1), 128), jnp.float32),
            pltpu.SemaphoreType.DMA((N_DEV - 1,)),
            pltpu.SemaphoreType.DMA((N_DEV - 1,)),
        ],
        compiler_params=pltpu.CompilerParams(collective_id=0),
    )(x, w_mat)


# device time: 99305 ns/iter; 3.5044x vs baseline; 1.0129x over previous
import jax
import jax.numpy as jnp
from jax import lax
from jax.experimental import pallas as pl
from jax.experimental.pallas import tpu as pltpu

N_DEV = 8


def kernel(x, w_mat):
    m_total, k_per = x.shape
    _, n = w_mat.shape
    m_per = m_total // N_DEV
    nh = n // 2

    def body(x_ref, w_ref, out_ref, commR, commL,
             sendR, recvR, sendL, recvL,
             amax_ref, amax_recv_ref, amax_send_sems, amax_recv_sems):
        my = lax.axis_index("i")
        left = lax.rem(my + N_DEV - 1, N_DEV)
        right = lax.rem(my + 1, N_DEV)

        barrier_sem = pltpu.get_barrier_semaphore()
        for nbr in (left, right):
            pl.semaphore_signal(
                barrier_sem, inc=1,
                device_id=(nbr,), device_id_type=pl.DeviceIdType.MESH,
            )
        pl.semaphore_wait(barrier_sem, 2)

        w_hi = w_ref[:, :].astype(jnp.bfloat16)
        w_lo = (w_ref[:, :] - w_hi.astype(jnp.float32)).astype(jnp.bfloat16)

        def dot3(x_blk, wh, wl):
            xh = x_blk.astype(jnp.bfloat16)
            xl = (x_blk - xh.astype(jnp.float32)).astype(jnp.bfloat16)
            acc = jnp.dot(xh, wh, preferred_element_type=jnp.float32)
            acc += jnp.dot(xh, wl, preferred_element_type=jnp.float32)
            acc += jnp.dot(xl, wh, preferred_element_type=jnp.float32)
            return acc

        nq = nh // 2

        def ring_rdma(comm, sems_send, sems_recv, s, c, nbr):
            return pltpu.make_async_remote_copy(
                src_ref=comm.at[s % 2, :, pl.ds(c * nq, nq)],
                dst_ref=comm.at[(s + 1) % 2, :, pl.ds(c * nq, nq)],
                send_sem=sems_send.at[s % 2, c],
                recv_sem=sems_recv.at[(s + 1) % 2, c],
                device_id=(nbr,),
                device_id_type=pl.DeviceIdType.MESH,
            )

        prevR = prevL = None
        yR = yL = None
        for s in range(N_DEV):
            slot = s % 2
            d_R = lax.rem(my + 2 * N_DEV - 1 - s, N_DEV)
            d_L = lax.rem(my + 1 + s, N_DEV)
            if s == 0:
                prevR = [ring_rdma(commR, sendR, recvR, s, c, right)
                         for c in (0, 1)]
                prevL = [ring_rdma(commL, sendL, recvL, s, c, left)
                         for c in (0, 1)]
                x_blk_R = x_ref[pl.ds(d_R * m_per, m_per), :]
                x_blk_L = x_ref[pl.ds(d_L * m_per, m_per), :]
                for c in (0, 1):
                    cols = pl.ds(c * nq, nq)
                    commR[slot, :, cols] = dot3(
                        x_blk_R, w_hi[:, nh + c * nq:nh + (c + 1) * nq],
                        w_lo[:, nh + c * nq:nh + (c + 1) * nq],
                    ).astype(jnp.bfloat16)
                    prevR[c].start()
                    commL[slot, :, cols] = dot3(
                        x_blk_L, w_hi[:, c * nq:(c + 1) * nq],
                        w_lo[:, c * nq:(c + 1) * nq],
                    ).astype(jnp.bfloat16)
                    prevL[c].start()
                continue
            pR = dot3(x_ref[pl.ds(d_R * m_per, m_per), :], w_hi[:, nh:],
                      w_lo[:, nh:])
            pL = dot3(x_ref[pl.ds(d_L * m_per, m_per), :], w_hi[:, :nh],
                      w_lo[:, :nh])
            if s < N_DEV - 1:
                nextR = [ring_rdma(commR, sendR, recvR, s, c, right)
                         for c in (0, 1)]
                nextL = [ring_rdma(commL, sendL, recvL, s, c, left)
                         for c in (0, 1)]
                for c in (0, 1):
                    cols = pl.ds(c * nq, nq)
                    prevR[c].wait()
                    commR[slot, :, cols] = (
                        commR[slot, :, cols].astype(jnp.float32)
                        + pR[:, c * nq:(c + 1) * nq]).astype(jnp.bfloat16)
                    nextR[c].start()
                    prevL[c].wait()
                    commL[slot, :, cols] = (
                        commL[slot, :, cols].astype(jnp.float32)
                        + pL[:, c * nq:(c + 1) * nq]).astype(jnp.bfloat16)
                    nextL[c].start()
                prevR, prevL = nextR, nextL
            else:
                prevR[0].wait()
                prevR[1].wait()
                yR = commR[slot].astype(jnp.float32) + pR
                prevL[0].wait()
                prevL[1].wait()
                yL = commL[slot].astype(jnp.float32) + pL

        local_amax = jnp.maximum(jnp.max(jnp.abs(yL)), jnp.max(jnp.abs(yR)))
        amax_ref[:, :] = jnp.full((8, 128), local_amax, dtype=jnp.float32)
        amax_rdmas = []
        for k in range(1, N_DEV):
            peer = lax.rem(my + k, N_DEV)
            rd = pltpu.make_async_remote_copy(
                src_ref=amax_ref.at[pl.ds(0, 8), :],
                dst_ref=amax_recv_ref.at[pl.ds((k - 1) * 8, 8), :],
                send_sem=amax_send_sems.at[k - 1],
                recv_sem=amax_recv_sems.at[k - 1],
                device_id=(peer,),
                device_id_type=pl.DeviceIdType.MESH,
            )
            rd.start()
            amax_rdmas.append(rd)
        for rd in amax_rdmas:
            rd.wait()
        global_amax = jnp.maximum(local_amax, jnp.max(amax_recv_ref[:, :]))

        scale = global_amax / 127.0
        qL = jnp.clip(jnp.round(yL / scale), -127.0, 127.0)
        qR = jnp.clip(jnp.round(yR / scale), -127.0, 127.0)
        out_ref[:, :nh] = qL * scale
        out_ref[:, nh:] = qR * scale

    return pl.pallas_call(
        body,
        out_shape=jax.ShapeDtypeStruct((m_per, n), jnp.float32),
        in_specs=[
            pl.BlockSpec(memory_space=pltpu.VMEM),
            pl.BlockSpec(memory_space=pltpu.VMEM),
        ],
        out_specs=pl.BlockSpec(memory_space=pltpu.VMEM),
        scratch_shapes=[
            pltpu.VMEM((2, m_per, nh), jnp.bfloat16),
            pltpu.VMEM((2, m_per, nh), jnp.bfloat16),
            pltpu.SemaphoreType.DMA((2, 2)),
            pltpu.SemaphoreType.DMA((2, 2)),
            pltpu.SemaphoreType.DMA((2, 2)),
            pltpu.SemaphoreType.DMA((2, 2)),
            pltpu.VMEM((8, 128), jnp.float32),
            pltpu.VMEM((8 * (N_DEV - 1), 128), jnp.float32),
            pltpu.SemaphoreType.DMA((N_DEV - 1,)),
            pltpu.SemaphoreType.DMA((N_DEV - 1,)),
        ],
        compiler_params=pltpu.CompilerParams(collective_id=0),
    )(x, w_mat)
